# baseline (speedup 1.0000x reference)
import jax
import jax.numpy as jnp
from jax import lax
from jax.experimental import pallas as pl
from jax.experimental.pallas import tpu as pltpu
from jax.experimental.pallas import tpu_sc as plsc

_NC = 2
_NS = 16
_NW = _NC * _NS
_L = 16
_CHUNK = 128


def _wmf_body(uid_hbm, iid_hbm, user_hbm, item_hbm, out_hbm,
              uidx_v, iidx_v, u_rows, v_rows, out_v, sem):
    wid = lax.axis_index("s") * _NC + lax.axis_index("c")
    n_chunks = uidx_v.shape[0]
    n_groups = out_v.shape[0]

    pltpu.sync_copy(uid_hbm.at[wid], uidx_v)
    pltpu.sync_copy(iid_hbm.at[wid], iidx_v)

    copies = []
    for j in range(n_chunks):
        copies.append(pltpu.async_copy(
            user_hbm.at[uidx_v.at[j]],
            u_rows.at[pl.ds(j * _CHUNK, _CHUNK)], sem))
        copies.append(pltpu.async_copy(
            item_hbm.at[iidx_v.at[j]],
            v_rows.at[pl.ds(j * _CHUNK, _CHUNK)], sem))
    for c in copies:
        c.wait()

    lanes = lax.iota(jnp.int32, _L)

    def group(g, carry):
        rows = g * _L + lanes
        acc = jnp.zeros((_L,), jnp.float32)
        for k in range(_L):
            col = jnp.full((_L,), k, jnp.int32)
            acc = acc + (plsc.load_gather(u_rows, [rows, col]) *
                         plsc.load_gather(v_rows, [rows, col]))
        out_v[g] = acc
        return carry

    lax.fori_loop(0, n_groups, group, 0)
    pltpu.sync_copy(out_v, out_hbm.at[wid])


def kernel(uid, iid, user_mat, item_mat):
    batch = uid.shape[0]
    n_chunks = batch // (_NW * _CHUNK)
    b_per_w = n_chunks * _CHUNK
    n_groups = b_per_w // _L

    uid3 = uid.astype(jnp.int32).reshape(_NW, n_chunks, _CHUNK)
    iid3 = iid.astype(jnp.int32).reshape(_NW, n_chunks, _CHUNK)

    f = pl.kernel(
        _wmf_body,
        out_type=jax.ShapeDtypeStruct((_NW, n_groups, _L), jnp.float32),
        mesh=plsc.VectorSubcoreMesh(core_axis_name="c", subcore_axis_name="s"),
        compiler_params=pltpu.CompilerParams(
            needs_layout_passes=False, use_tc_tiling_on_sc=False),
        scratch_types=[
            pltpu.VMEM((n_chunks, _CHUNK), jnp.int32),
            pltpu.VMEM((n_chunks, _CHUNK), jnp.int32),
            pltpu.VMEM((b_per_w, _L), jnp.float32),
            pltpu.VMEM((b_per_w, _L), jnp.float32),
            pltpu.VMEM((n_groups, _L), jnp.float32),
            pltpu.SemaphoreType.DMA,
        ],
    )
    out = f(uid3, iid3, user_mat, item_mat)
    return out.reshape(batch)
